# Initial kernel scaffold; baseline (speedup 1.0000x reference)
import jax
import jax.numpy as jnp
from jax.experimental import pallas as pl


def kernel(clean_ids, diff_logits, t, mask_noise):
    raise NotImplementedError("write your pallas kernel here")


# speedup vs baseline: 3.3324x; 3.3324x over previous
import functools
import math

import jax
import jax.numpy as jnp
from jax.experimental import pallas as pl
from jax.experimental.pallas import tpu as pltpu

MASK_TOKEN_ID = 31999
PAD_TOKEN_ID = 0
DT = 1e-05

ROWS_BLK = 128


def _loss_kernel(x_ref, ids_ref, noise_ref, p_ref, w_ref, out_ref,
                 acc_num, acc_den, *, n_steps):
    pid = pl.program_id(0)

    @pl.when(pid == 0)
    def _init():
        acc_num[...] = jnp.zeros_like(acc_num)
        acc_den[...] = jnp.zeros_like(acc_den)

    x = x_ref[...]
    ids = ids_ref[...]
    m = jnp.max(x, axis=1, keepdims=True)
    s = jnp.sum(jnp.exp(x - m), axis=1, keepdims=True)
    lse = m + jnp.log(s)
    cols = jax.lax.broadcasted_iota(jnp.int32, x.shape, 1)
    label_logit = jnp.sum(jnp.where(cols == ids, x, 0.0), axis=1,
                          keepdims=True)
    nll = lse - label_logit
    maskf = jnp.where((noise_ref[...] < p_ref[...]) & (ids != PAD_TOKEN_ID),
                      1.0, 0.0)
    acc_num[...] += jnp.sum(nll * w_ref[...] * maskf).reshape(1, 1)
    acc_den[...] += jnp.sum(maskf).reshape(1, 1)

    @pl.when(pid == n_steps - 1)
    def _fin():
        out_ref[...] = acc_num[...] / jnp.maximum(acc_den[...], 1.0)


def kernel(clean_ids, diff_logits, t, mask_noise):
    B, T, V = diff_logits.shape
    N = B * T
    n_steps = N // ROWS_BLK

    a_t = jnp.cos(0.5 * math.pi * t)
    a_tp = jnp.cos(0.5 * math.pi * jnp.minimum(t + DT, 1.0))
    p_mask = 1.0 - a_t
    weights = jnp.maximum(jnp.abs(a_tp - a_t) / DT, 1e-6)

    x2 = diff_logits.reshape(N, V)
    ids2 = clean_ids.reshape(N, 1).astype(jnp.int32)
    noise2 = mask_noise.reshape(N, 1)
    p2 = jnp.broadcast_to(p_mask[:, None], (B, T)).reshape(N, 1)
    w2 = jnp.broadcast_to(weights[:, None], (B, T)).reshape(N, 1)

    row_spec = pl.BlockSpec((ROWS_BLK, 1), lambda i: (i, 0))
    out = pl.pallas_call(
        functools.partial(_loss_kernel, n_steps=n_steps),
        grid=(n_steps,),
        in_specs=[
            pl.BlockSpec((ROWS_BLK, V), lambda i: (i, 0)),
            row_spec, row_spec, row_spec, row_spec,
        ],
        out_specs=pl.BlockSpec((1, 1), lambda i: (0, 0)),
        out_shape=jax.ShapeDtypeStruct((1, 1), jnp.float32),
        scratch_shapes=[
            pltpu.VMEM((1, 1), jnp.float32),
            pltpu.VMEM((1, 1), jnp.float32),
        ],
    )(x2, ids2, noise2, p2, w2)
    return out.reshape(())
